# baseline (speedup 1.0000x reference)
import functools

import jax
import jax.numpy as jnp
from jax import lax
from jax.experimental import pallas as pl
from jax.experimental.pallas import tpu as pltpu
from jax.experimental.pallas import tpu_sc as plsc

_SIZE = 1_000_000
_DIM = 32
_BATCH = 4096
_NSAMP = 50
_NCAND = _NSAMP - 1
_EPS = 1e-5

_NC, _NS, _L = 2, 16, 16
_NW = _NC * _NS
_ROWS_W = _BATCH // _NW
_CHUNK = 16
_NCHUNK = _ROWS_W // _CHUNK
_NIDX = _CHUNK * _NSAMP
_GROUPS = 4
_OUTP = _GROUPS * _L

_GATHER_SPLITS = [128] * (_NIDX // 128) + ([_NIDX % 128] if _NIDX % 128 else [])


def _sc_body(idx_hbm, w_hbm, x_hbm, idx_v, rows_v, out_v, sem):
    wid = lax.axis_index("s") * _NC + lax.axis_index("c")
    iota = lax.iota(jnp.int32, _L)

    def do_chunk(c, carry):
        row0 = wid * _ROWS_W + c * _CHUNK
        flat0 = row0 * _NSAMP
        pltpu.sync_copy(idx_hbm.at[pl.ds(flat0, _NIDX)], idx_v)

        handles = []
        off = 0
        for sz in _GATHER_SPLITS:
            handles.append(
                pltpu.async_copy(
                    w_hbm.at[idx_v.at[pl.ds(off, sz)]],
                    rows_v.at[pl.ds(off, sz), :],
                    sem,
                )
            )
            off += sz
        for h in handles:
            h.wait()

        def do_row(b, inner):
            rbase = b * _NSAMP
            ridx = []
            for g in range(_GROUPS):
                r = rbase + 1 + g * _L + iota
                if g == _GROUPS - 1:
                    r = jnp.minimum(r, rbase + _NCAND)
                ridx.append(r)
            s_lo = rows_v[rbase, pl.ds(0, _L)]
            s_hi = rows_v[rbase, pl.ds(_L, _L)]
            accs = [None] * _GROUPS
            for d in range(_DIM):
                s = s_lo[d] if d < _L else s_hi[d - _L]
                cv = jnp.full((_L,), s, jnp.float32)
                col = jnp.full((_L,), d, jnp.int32)
                for g in range(_GROUPS):
                    gv = plsc.load_gather(rows_v, [ridx[g], col])
                    if d == 0:
                        accs[g] = gv * cv
                    else:
                        accs[g] = accs[g] - gv * cv
            for g in range(_GROUPS):
                out_v[b, pl.ds(g * _L, _L)] = accs[g]
            return inner

        lax.fori_loop(0, _CHUNK, do_row, 0)
        pltpu.sync_copy(out_v, x_hbm.at[pl.ds(row0, _CHUNK), :])
        return carry

    lax.fori_loop(0, _NCHUNK, do_chunk, 0)


_sc_gather_dot = functools.partial(
    pl.kernel,
    out_type=jax.ShapeDtypeStruct((_BATCH, _OUTP), jnp.float32),
    mesh=plsc.VectorSubcoreMesh(
        core_axis_name="c", subcore_axis_name="s", num_cores=_NC, num_subcores=_NS
    ),
    scratch_types=[
        pltpu.VMEM((_NIDX,), jnp.int32),
        pltpu.VMEM((_NIDX, _DIM), jnp.float32),
        pltpu.VMEM((_CHUNK, _OUTP), jnp.float32),
        pltpu.SemaphoreType.DMA,
    ],
    compiler_params=pltpu.CompilerParams(
        needs_layout_passes=False, use_tc_tiling_on_sc=False
    ),
)(_sc_body)


def _tc_finish_body(x_ref, o_ref):
    x = x_ref[...][:, :_NCAND]
    x = jnp.maximum(x, 1.0 + _EPS)
    o_ref[...] = -jnp.log(x + jnp.sqrt((x - 1.0) * (x + 1.0)))


def _tc_finish(x):
    blk = 512
    return pl.pallas_call(
        _tc_finish_body,
        grid=(_BATCH // blk,),
        in_specs=[pl.BlockSpec((blk, _OUTP), lambda i: (i, 0))],
        out_specs=pl.BlockSpec((blk, _NCAND), lambda i: (i, 0)),
        out_shape=jax.ShapeDtypeStruct((_BATCH, _NCAND), jnp.float32),
    )(x)


def kernel(inputs, weight):
    idx_flat = inputs.reshape(-1)
    x = _sc_gather_dot(idx_flat, weight)
    return _tc_finish(x)
